# baseline (speedup 1.0000x reference)
import jax
import jax.numpy as jnp
from jax.experimental import pallas as pl
from jax.experimental.pallas import tpu as pltpu

_BLK = 1024


def _body(row_off, nv_ref, np_ref, rp_ref, uc_ref, u_ref, out_ref):
    i = pl.program_id(0)
    j = pl.program_id(1)

    k_sizes = np_ref.shape[1]
    total = np_ref[0, 0]
    for k in range(1, k_sizes):
        total = total + np_ref[0, k]
    u = uc_ref[0]
    idx = 0
    csums = []
    acc = None
    for k in range(k_sizes):
        p = np_ref[0, k] / total
        acc = p if acc is None else acc + p
        csums.append(acc)
    for k in range(k_sizes - 1, -1, -1):
        idx = jnp.where(csums[k] >= u, k, idx)
    n_nodes = nv_ref[0, idx]
    r = rp_ref[0, idx]

    blk = out_ref.shape[0]
    row = (row_off * blk + i * blk
           + jax.lax.broadcasted_iota(jnp.int32, out_ref.shape, 0)).astype(
               jnp.float32)
    col = (j * blk + jax.lax.broadcasted_iota(jnp.int32, out_ref.shape, 1)
           ).astype(jnp.float32)

    u2 = u_ref[...]
    uval = jnp.where(col > row, u2, u2.T)
    link = (uval <= r) & (col != row) & (row < n_nodes) & (col < n_nodes)
    out_ref[...] = link.astype(jnp.int32)


def _band(row_off, gr, gc, nv, npr, rp, uc, u_links):
    import functools
    n = u_links.shape[1]

    def u_map(i, j):
        gi = i + row_off
        return (jnp.minimum(gi, j), jnp.maximum(gi, j))

    return pl.pallas_call(
        functools.partial(_body, row_off),
        grid=(gr, gc),
        in_specs=[
            pl.BlockSpec(memory_space=pltpu.SMEM),
            pl.BlockSpec(memory_space=pltpu.SMEM),
            pl.BlockSpec(memory_space=pltpu.SMEM),
            pl.BlockSpec(memory_space=pltpu.SMEM),
            pl.BlockSpec((_BLK, _BLK), u_map),
        ],
        out_specs=pl.BlockSpec((_BLK, _BLK), lambda i, j: (i, j)),
        out_shape=jax.ShapeDtypeStruct((gr * _BLK, n), jnp.int32),
    )(nv, npr, rp, uc, u_links)


def kernel(N_values, N_probs, r_probs, u_cat, u_links):
    n = u_links.shape[0]
    g = n // _BLK
    gtop = g // 2
    nv = N_values.reshape(1, 16)
    npr = N_probs.reshape(1, 16)
    rp = r_probs.reshape(1, 16)
    uc = u_cat.reshape(1)
    top = _band(0, gtop, g, nv, npr, rp, uc, u_links)
    bot = _band(gtop, g - gtop, g, nv, npr, rp, uc, u_links)
    return jnp.concatenate([top, bot], axis=0)


# speedup vs baseline: 1.7876x; 1.7876x over previous
import jax
import jax.numpy as jnp
import numpy as np
from jax.experimental import pallas as pl
from jax.experimental.pallas import tpu as pltpu

_BLK = 512


def _sample(nv_ref, np_ref, rp_ref, uc_ref):
    k_sizes = np_ref.shape[1]
    total = np_ref[0, 0]
    for k in range(1, k_sizes):
        total = total + np_ref[0, k]
    u = uc_ref[0]
    idx = 0
    csums = []
    acc = None
    for k in range(k_sizes):
        p = np_ref[0, k] / total
        acc = p if acc is None else acc + p
        csums.append(acc)
    for k in range(k_sizes - 1, -1, -1):
        idx = jnp.where(csums[k] >= u, k, idx)
    return nv_ref[0, idx], rp_ref[0, idx]


def _make_body(num_steps, blk):
    def _body(bi_ref, bj_ref, nv_ref, np_ref, rp_ref, uc_ref, u_ref, out_ref,
              obuf, tbuf, sems):
        k = pl.program_id(0)
        slot = jax.lax.rem(k, 2)
        bi = bi_ref[k]
        bj = bj_ref[k]

        def _wait_step(step, wslot):
            pb = bi_ref[step]
            qb = bj_ref[step]
            pltpu.make_async_copy(
                obuf.at[wslot],
                out_ref.at[pl.ds(pb * blk, blk), pl.ds(qb * blk, blk)],
                sems.at[wslot, 0],
            ).wait()

            @pl.when(pb != qb)
            def _():
                pltpu.make_async_copy(
                    tbuf.at[wslot],
                    out_ref.at[pl.ds(qb * blk, blk), pl.ds(pb * blk, blk)],
                    sems.at[wslot, 1],
                ).wait()

        @pl.when(k >= 2)
        def _():
            _wait_step(k - 2, slot)

        n_nodes, r = _sample(nv_ref, np_ref, rp_ref, uc_ref)

        row = (bi * blk + jax.lax.broadcasted_iota(jnp.int32, (blk, blk), 0)
               ).astype(jnp.float32)
        col = (bj * blk + jax.lax.broadcasted_iota(jnp.int32, (blk, blk), 1)
               ).astype(jnp.float32)
        u = u_ref[...]
        lu = ((u <= r) & (col > row) & (row < n_nodes) & (col < n_nodes)
              ).astype(jnp.int32)
        lt = lu.T

        @pl.when(bi == bj)
        def _():
            obuf[slot] = lu | lt

        @pl.when(bi != bj)
        def _():
            obuf[slot] = lu
            tbuf[slot] = lt

        pltpu.make_async_copy(
            obuf.at[slot],
            out_ref.at[pl.ds(bi * blk, blk), pl.ds(bj * blk, blk)],
            sems.at[slot, 0],
        ).start()

        @pl.when(bi != bj)
        def _():
            pltpu.make_async_copy(
                tbuf.at[slot],
                out_ref.at[pl.ds(bj * blk, blk), pl.ds(bi * blk, blk)],
                sems.at[slot, 1],
            ).start()

        @pl.when(k == num_steps - 1)
        def _():
            if num_steps >= 2:
                _wait_step(num_steps - 2, jax.lax.rem(num_steps - 2, 2))
            _wait_step(num_steps - 1, jax.lax.rem(num_steps - 1, 2))

    return _body


def kernel(N_values, N_probs, r_probs, u_cat, u_links):
    n = u_links.shape[0]
    g = n // _BLK
    pairs = [(i, j) for i in range(g) for j in range(i, g)]
    num_steps = len(pairs)
    bi_arr = jnp.asarray(np.array([p[0] for p in pairs], dtype=np.int32))
    bj_arr = jnp.asarray(np.array([p[1] for p in pairs], dtype=np.int32))

    grid_spec = pltpu.PrefetchScalarGridSpec(
        num_scalar_prefetch=6,
        grid=(num_steps,),
        in_specs=[
            pl.BlockSpec((_BLK, _BLK), lambda k, bi, bj, nv, npr, rp, uc:
                         (bi[k], bj[k])),
        ],
        out_specs=pl.BlockSpec(memory_space=pl.ANY),
        scratch_shapes=[
            pltpu.VMEM((2, _BLK, _BLK), jnp.int32),
            pltpu.VMEM((2, _BLK, _BLK), jnp.int32),
            pltpu.SemaphoreType.DMA((2, 2)),
        ],
    )

    return pl.pallas_call(
        _make_body(num_steps, _BLK),
        grid_spec=grid_spec,
        out_shape=jax.ShapeDtypeStruct((n, n), jnp.int32),
    )(
        bi_arr,
        bj_arr,
        N_values.reshape(1, 16),
        N_probs.reshape(1, 16),
        r_probs.reshape(1, 16),
        u_cat.reshape(1),
        u_links,
    )


# speedup vs baseline: 2.4163x; 1.3517x over previous
import jax
import jax.numpy as jnp
import numpy as np
from jax.experimental import pallas as pl
from jax.experimental.pallas import tpu as pltpu

_BLK = 1024


def _sample(nv_ref, np_ref, rp_ref, uc_ref):
    k_sizes = np_ref.shape[1]
    total = np_ref[0, 0]
    for k in range(1, k_sizes):
        total = total + np_ref[0, k]
    u = uc_ref[0]
    idx = 0
    csums = []
    acc = None
    for k in range(k_sizes):
        p = np_ref[0, k] / total
        acc = p if acc is None else acc + p
        csums.append(acc)
    for k in range(k_sizes - 1, -1, -1):
        idx = jnp.where(csums[k] >= u, k, idx)
    return nv_ref[0, idx], rp_ref[0, idx]


_NBUF = 3


def _make_body(num_steps, blk):
    def _body(bi_ref, bj_ref, nv_ref, np_ref, rp_ref, uc_ref, u_ref, out_ref,
              obuf, tbuf, sems):
        k = pl.program_id(0)
        slot = jax.lax.rem(k, _NBUF)
        bi = bi_ref[k]
        bj = bj_ref[k]

        def _wait_step(step, wslot):
            pb = bi_ref[step]
            qb = bj_ref[step]
            pltpu.make_async_copy(
                obuf.at[wslot],
                out_ref.at[pl.ds(pb * blk, blk), pl.ds(qb * blk, blk)],
                sems.at[wslot, 0],
            ).wait()

            @pl.when(pb != qb)
            def _():
                pltpu.make_async_copy(
                    tbuf.at[wslot],
                    out_ref.at[pl.ds(qb * blk, blk), pl.ds(pb * blk, blk)],
                    sems.at[wslot, 1],
                ).wait()

        @pl.when(k >= _NBUF)
        def _():
            _wait_step(k - _NBUF, slot)

        n_nodes, r = _sample(nv_ref, np_ref, rp_ref, uc_ref)

        row = (bi * blk + jax.lax.broadcasted_iota(jnp.int32, (blk, blk), 0)
               ).astype(jnp.float32)
        col = (bj * blk + jax.lax.broadcasted_iota(jnp.int32, (blk, blk), 1)
               ).astype(jnp.float32)
        u = u_ref[...]
        lu = ((u <= r) & (col > row) & (row < n_nodes) & (col < n_nodes)
              ).astype(jnp.int32)
        lt = lu.T

        @pl.when(bi == bj)
        def _():
            obuf[slot] = lu | lt

        @pl.when(bi != bj)
        def _():
            obuf[slot] = lu
            tbuf[slot] = lt

        pltpu.make_async_copy(
            obuf.at[slot],
            out_ref.at[pl.ds(bi * blk, blk), pl.ds(bj * blk, blk)],
            sems.at[slot, 0],
        ).start()

        @pl.when(bi != bj)
        def _():
            pltpu.make_async_copy(
                tbuf.at[slot],
                out_ref.at[pl.ds(bj * blk, blk), pl.ds(bi * blk, blk)],
                sems.at[slot, 1],
            ).start()

        @pl.when(k == num_steps - 1)
        def _():
            for s in range(max(0, num_steps - _NBUF), num_steps):
                _wait_step(s, s % _NBUF)

    return _body


def kernel(N_values, N_probs, r_probs, u_cat, u_links):
    n = u_links.shape[0]
    g = n // _BLK
    pairs = [(i, j) for i in range(g) for j in range(i, g)]
    num_steps = len(pairs)
    bi_arr = jnp.asarray(np.array([p[0] for p in pairs], dtype=np.int32))
    bj_arr = jnp.asarray(np.array([p[1] for p in pairs], dtype=np.int32))

    grid_spec = pltpu.PrefetchScalarGridSpec(
        num_scalar_prefetch=6,
        grid=(num_steps,),
        in_specs=[
            pl.BlockSpec((_BLK, _BLK), lambda k, bi, bj, nv, npr, rp, uc:
                         (bi[k], bj[k])),
        ],
        out_specs=pl.BlockSpec(memory_space=pl.ANY),
        scratch_shapes=[
            pltpu.VMEM((_NBUF, _BLK, _BLK), jnp.int32),
            pltpu.VMEM((_NBUF, _BLK, _BLK), jnp.int32),
            pltpu.SemaphoreType.DMA((_NBUF, 2)),
        ],
    )

    return pl.pallas_call(
        _make_body(num_steps, _BLK),
        grid_spec=grid_spec,
        out_shape=jax.ShapeDtypeStruct((n, n), jnp.int32),
    )(
        bi_arr,
        bj_arr,
        N_values.reshape(1, 16),
        N_probs.reshape(1, 16),
        r_probs.reshape(1, 16),
        u_cat.reshape(1),
        u_links,
    )
